# baseline (speedup 1.0000x reference)
import jax
import jax.numpy as jnp
from jax.experimental import pallas as pl
from jax.experimental.pallas import tpu as pltpu


def _stgcn_forward(x, a_hat, w1m, b1, w2m, b2, theta, gamma, beta):
    B, N, T, Cin = x.shape
    K1 = 3
    T1 = T - K1 + 1
    Cout = w1m.shape[-1] // 2
    Csp = theta.shape[1]
    K2 = 3
    T2 = T1 - K2 + 1
    Cout2 = w2m.shape[1] // 2

    w1b = w1m.astype(jnp.bfloat16)
    w2b = w2m.astype(jnp.bfloat16)
    thb = theta.astype(jnp.bfloat16)
    ab = a_hat.astype(jnp.bfloat16)

    BB = 4
    while B % BB:
        BB //= 2

    xcat = jnp.concatenate([x[:, :, k:k + T1, :] for k in range(K1)],
                           axis=-1).astype(jnp.bfloat16)
    xcat = xcat.reshape(B, N * T1, K1 * Cin)

    def k1(x_ref, w_ref, b_ref, th_ref, a_ref, o_ref):
        xm = x_ref[...].reshape(BB * N * T1, K1 * Cin)
        y = jnp.dot(xm, w_ref[...],
                    preferred_element_type=jnp.float32) + b_ref[...]
        t = jnp.maximum(y[:, :Cout] + jax.nn.sigmoid(y[:, Cout:]), 0.0)
        u = jnp.dot(t.astype(jnp.bfloat16), th_ref[...],
                    preferred_element_type=jnp.float32)
        u2 = u.astype(jnp.bfloat16).reshape(BB, N, T1 * Csp)
        for i in range(BB):
            lfs = jnp.dot(a_ref[...], u2[i],
                          preferred_element_type=jnp.float32)
            o_ref[i] = jnp.maximum(lfs, 0.0).astype(jnp.bfloat16)

    t2 = pl.pallas_call(
        k1,
        grid=(B // BB,),
        in_specs=[
            pl.BlockSpec((BB, N * T1, K1 * Cin), lambda b: (b, 0, 0)),
            pl.BlockSpec((K1 * Cin, 2 * Cout), lambda b: (0, 0)),
            pl.BlockSpec((1, 2 * Cout), lambda b: (0, 0)),
            pl.BlockSpec((Cout, Csp), lambda b: (0, 0)),
            pl.BlockSpec((N, N), lambda b: (0, 0)),
        ],
        out_specs=pl.BlockSpec((BB, N, T1 * Csp), lambda b: (b, 0, 0)),
        out_shape=jax.ShapeDtypeStruct((B, N, T1 * Csp), jnp.bfloat16),
        compiler_params=pltpu.CompilerParams(dimension_semantics=("parallel",)),
    )(xcat, w1b, b1, thb, ab)

    NB = 16
    while N % NB:
        NB //= 2

    def k2(t2_ref, w_ref, b_ref, g_ref, bt_ref, o_ref):
        n0 = pl.program_id(0)
        z = t2_ref[...].reshape(B * NB, T1, Csp)
        zcat = jnp.concatenate([z[:, k:k + T2, :] for k in range(K2)],
                               axis=-1)
        zm = zcat.reshape(B * NB * T2, K2 * Csp)
        y = jnp.dot(zm, w_ref[...],
                    preferred_element_type=jnp.float32) + b_ref[...]
        t3 = jnp.maximum(y[:, :Cout2] + jax.nn.sigmoid(y[:, Cout2:]), 0.0)
        t4 = t3.reshape(B, NB, T2, Cout2)
        mean = jnp.mean(t4, axis=(0, 2, 3), keepdims=True)
        cen = t4 - mean
        var = jnp.mean(cen * cen, axis=(0, 2, 3), keepdims=True)
        inv = jax.lax.rsqrt(var + 1e-5)
        for i in range(NB):
            g = g_ref[n0 * NB + i]
            be = bt_ref[n0 * NB + i]
            o_ref[:, i] = cen[:, i] * (inv[0, i] * g) + be

    out = pl.pallas_call(
        k2,
        grid=(N // NB,),
        in_specs=[
            pl.BlockSpec((B, NB, T1 * Csp), lambda n: (0, n, 0)),
            pl.BlockSpec((K2 * Csp, 2 * Cout2), lambda n: (0, 0)),
            pl.BlockSpec((1, 2 * Cout2), lambda n: (0, 0)),
            pl.BlockSpec(memory_space=pltpu.MemorySpace.SMEM),
            pl.BlockSpec(memory_space=pltpu.MemorySpace.SMEM),
        ],
        out_specs=pl.BlockSpec((B, NB, T2, Cout2), lambda n: (0, n, 0, 0)),
        out_shape=jax.ShapeDtypeStruct((B, N, T2, Cout2), jnp.float32),
        compiler_params=pltpu.CompilerParams(dimension_semantics=("parallel",)),
    )(t2, w2b, b2, gamma, beta)
    return out


def kernel(x, a_hat, w11, b11, w12, b12, w13, b13,
           w21, b21, w22, b22, w23, b23, theta, gamma, beta):
    K1, Cin, Cout = w11.shape
    K2, Csp, Cout2 = w21.shape
    w1m = jnp.concatenate([w11 + w13, w12], axis=-1).reshape(K1 * Cin, 2 * Cout)
    b1 = jnp.concatenate([b11 + b13, b12], axis=-1)
    w2m = jnp.concatenate([w21 + w23, w22], axis=-1).reshape(K2 * Csp, 2 * Cout2)
    b2 = jnp.concatenate([b21 + b23, b22], axis=-1)
    return _stgcn_forward(x, a_hat, w1m, b1, w2m, b2, theta, gamma, beta)


# speedup vs baseline: 1.7397x; 1.7397x over previous
import jax
import jax.numpy as jnp
from jax.experimental import pallas as pl
from jax.experimental.pallas import tpu as pltpu


def _stgcn_forward(x, a_hat, w1m, b1, w2m, b2, theta, gamma, beta):
    B, N, T, Cin = x.shape
    K1 = 3
    T1 = T - K1 + 1
    Cout = w1m.shape[-1] // 2
    Csp = theta.shape[1]
    K2 = 3
    T2 = T1 - K2 + 1
    Cout2 = w2m.shape[1] // 2

    w1b = w1m.astype(jnp.bfloat16)
    w2b = w2m.astype(jnp.bfloat16)
    thb = theta.astype(jnp.bfloat16)
    ab = a_hat.astype(jnp.bfloat16)

    BB = 4
    while B % BB:
        BB //= 2

    def k1(x_ref, w_ref, b_ref, th_ref, a_ref, o_ref):
        xk = x_ref[...].reshape(BB * N, T * Cin).astype(jnp.bfloat16)
        u_parts = []
        for t in range(T1):
            xs = xk[:, t * Cin:(t + K1) * Cin]
            y = jnp.dot(xs, w_ref[...],
                        preferred_element_type=jnp.float32) + b_ref[...]
            tt = jnp.maximum(y[:, :Cout] + jax.nn.sigmoid(y[:, Cout:]), 0.0)
            u_parts.append(jnp.dot(tt.astype(jnp.bfloat16), th_ref[...],
                                   preferred_element_type=jnp.float32))
        u_all = jnp.concatenate(u_parts, axis=-1)
        u2 = u_all.astype(jnp.bfloat16).reshape(BB, N, T1 * Csp)
        for i in range(BB):
            lfs = jnp.dot(a_ref[...], u2[i],
                          preferred_element_type=jnp.float32)
            o_ref[i] = jnp.maximum(lfs, 0.0).astype(jnp.bfloat16)

    t2 = pl.pallas_call(
        k1,
        grid=(B // BB,),
        in_specs=[
            pl.BlockSpec((BB, N, T * Cin), lambda b: (b, 0, 0)),
            pl.BlockSpec((K1 * Cin, 2 * Cout), lambda b: (0, 0)),
            pl.BlockSpec((1, 2 * Cout), lambda b: (0, 0)),
            pl.BlockSpec((Cout, Csp), lambda b: (0, 0)),
            pl.BlockSpec((N, N), lambda b: (0, 0)),
        ],
        out_specs=pl.BlockSpec((BB, N, T1 * Csp), lambda b: (b, 0, 0)),
        out_shape=jax.ShapeDtypeStruct((B, N, T1 * Csp), jnp.bfloat16),
        compiler_params=pltpu.CompilerParams(dimension_semantics=("parallel",)),
    )(x.reshape(B, N, T * Cin), w1b, b1, thb, ab)

    NB = 16
    while N % NB:
        NB //= 2

    def k2(t2_ref, w_ref, b_ref, g_ref, bt_ref, o_ref):
        n0 = pl.program_id(0)
        z = t2_ref[...].reshape(B * NB, T1 * Csp)
        parts = []
        for t in range(T2):
            zs = z[:, t * Csp:(t + K2) * Csp]
            y = jnp.dot(zs, w_ref[...],
                        preferred_element_type=jnp.float32) + b_ref[...]
            parts.append(
                jnp.maximum(y[:, :Cout2] + jax.nn.sigmoid(y[:, Cout2:]), 0.0))
        t3 = jnp.concatenate(parts, axis=-1)
        t4 = t3.reshape(B, NB, T2 * Cout2)
        mean = jnp.mean(t4, axis=(0, 2), keepdims=True)
        cen = t4 - mean
        var = jnp.mean(cen * cen, axis=(0, 2), keepdims=True)
        inv = jax.lax.rsqrt(var + 1e-5)
        gvec = jnp.stack([g_ref[n0 * NB + i] for i in range(NB)])
        bvec = jnp.stack([bt_ref[n0 * NB + i] for i in range(NB)])
        scale = inv * gvec.reshape(1, NB, 1)
        o_ref[...] = cen * scale + bvec.reshape(1, NB, 1)

    out = pl.pallas_call(
        k2,
        grid=(N // NB,),
        in_specs=[
            pl.BlockSpec((B, NB, T1 * Csp), lambda n: (0, n, 0)),
            pl.BlockSpec((K2 * Csp, 2 * Cout2), lambda n: (0, 0)),
            pl.BlockSpec((1, 2 * Cout2), lambda n: (0, 0)),
            pl.BlockSpec(memory_space=pltpu.MemorySpace.SMEM),
            pl.BlockSpec(memory_space=pltpu.MemorySpace.SMEM),
        ],
        out_specs=pl.BlockSpec((B, NB, T2 * Cout2), lambda n: (0, n, 0)),
        out_shape=jax.ShapeDtypeStruct((B, N, T2 * Cout2), jnp.float32),
        compiler_params=pltpu.CompilerParams(dimension_semantics=("parallel",)),
    )(t2, w2b, b2, gamma, beta)
    return out.reshape(B, N, T2, Cout2)


def kernel(x, a_hat, w11, b11, w12, b12, w13, b13,
           w21, b21, w22, b22, w23, b23, theta, gamma, beta):
    K1, Cin, Cout = w11.shape
    K2, Csp, Cout2 = w21.shape
    w1m = jnp.concatenate([w11 + w13, w12], axis=-1).reshape(K1 * Cin, 2 * Cout)
    b1 = jnp.concatenate([b11 + b13, b12], axis=-1)
    w2m = jnp.concatenate([w21 + w23, w22], axis=-1).reshape(K2 * Csp, 2 * Cout2)
    b2 = jnp.concatenate([b21 + b23, b22], axis=-1)
    return _stgcn_forward(x, a_hat, w1m, b1, w2m, b2, theta, gamma, beta)


# speedup vs baseline: 1.9916x; 1.1448x over previous
import jax
import jax.numpy as jnp
from jax.experimental import pallas as pl
from jax.experimental.pallas import tpu as pltpu


def _stgcn_forward(x, a_hat, w1m, b1, w2m, b2, theta, gamma, beta):
    B, N, T, Cin = x.shape
    K1 = 3
    T1 = T - K1 + 1
    Cout = w1m.shape[-1] // 2
    Csp = theta.shape[1]
    K2 = 3
    T2 = T1 - K2 + 1
    Cout2 = w2m.shape[1] // 2

    w1b = w1m.astype(jnp.bfloat16)
    w2b = w2m.astype(jnp.bfloat16)
    ab = a_hat.astype(jnp.bfloat16)
    th2 = jnp.zeros((2 * Cout, 2 * Csp), theta.dtype)
    th2 = th2.at[:Cout, :Csp].set(theta).at[Cout:, Csp:].set(theta)
    th2b = th2.astype(jnp.bfloat16)

    BB = 4
    while B % BB:
        BB //= 2

    def k1(x_ref, w_ref, b_ref, th_ref, a_ref, o_ref):
        xk = x_ref[...].reshape(BB * N, T * Cin).astype(jnp.bfloat16)
        t_parts = []
        for t in range(T1):
            xs = xk[:, t * Cin:(t + K1) * Cin]
            y = jnp.dot(xs, w_ref[...],
                        preferred_element_type=jnp.float32) + b_ref[...]
            tt = jnp.maximum(y[:, :Cout] + jax.nn.sigmoid(y[:, Cout:]), 0.0)
            t_parts.append(tt.astype(jnp.bfloat16))
        u_parts = []
        for p in range(0, T1, 2):
            pair = jnp.concatenate(t_parts[p:p + 2], axis=-1)
            u_parts.append(jnp.dot(pair, th_ref[...],
                                   preferred_element_type=jnp.float32))
        u_all = jnp.concatenate(u_parts, axis=-1)
        u2 = u_all.astype(jnp.bfloat16).reshape(BB, N, T1 * Csp)
        for i in range(BB):
            lfs = jnp.dot(a_ref[...], u2[i],
                          preferred_element_type=jnp.float32)
            o_ref[i] = jnp.maximum(lfs, 0.0).astype(jnp.bfloat16)

    t2 = pl.pallas_call(
        k1,
        grid=(B // BB,),
        in_specs=[
            pl.BlockSpec((BB, N, T * Cin), lambda b: (b, 0, 0)),
            pl.BlockSpec((K1 * Cin, 2 * Cout), lambda b: (0, 0)),
            pl.BlockSpec((1, 2 * Cout), lambda b: (0, 0)),
            pl.BlockSpec((2 * Cout, 2 * Csp), lambda b: (0, 0)),
            pl.BlockSpec((N, N), lambda b: (0, 0)),
        ],
        out_specs=pl.BlockSpec((BB, N, T1 * Csp), lambda b: (b, 0, 0)),
        out_shape=jax.ShapeDtypeStruct((B, N, T1 * Csp), jnp.bfloat16),
        compiler_params=pltpu.CompilerParams(dimension_semantics=("parallel",)),
    )(x.reshape(B, N, T * Cin), w1b, b1, th2b, ab)

    NB = 16
    while N % NB:
        NB //= 2

    def k2(t2_ref, w_ref, b_ref, g_ref, bt_ref, o_ref):
        n0 = pl.program_id(0)
        z = t2_ref[...].reshape(B * NB, T1 * Csp)
        parts = []
        for t in range(T2):
            zs = z[:, t * Csp:(t + K2) * Csp]
            y = jnp.dot(zs, w_ref[...],
                        preferred_element_type=jnp.float32) + b_ref[...]
            parts.append(
                jnp.maximum(y[:, :Cout2] + jax.nn.sigmoid(y[:, Cout2:]), 0.0))
        t3 = jnp.concatenate(parts, axis=-1)
        t4 = t3.reshape(B, NB, T2 * Cout2)
        mean = jnp.mean(t4, axis=(0, 2), keepdims=True)
        cen = t4 - mean
        var = jnp.mean(cen * cen, axis=(0, 2), keepdims=True)
        inv = jax.lax.rsqrt(var + 1e-5)
        gvec = jnp.stack([g_ref[n0 * NB + i] for i in range(NB)])
        bvec = jnp.stack([bt_ref[n0 * NB + i] for i in range(NB)])
        scale = inv * gvec.reshape(1, NB, 1)
        o_ref[...] = cen * scale + bvec.reshape(1, NB, 1)

    out = pl.pallas_call(
        k2,
        grid=(N // NB,),
        in_specs=[
            pl.BlockSpec((B, NB, T1 * Csp), lambda n: (0, n, 0)),
            pl.BlockSpec((K2 * Csp, 2 * Cout2), lambda n: (0, 0)),
            pl.BlockSpec((1, 2 * Cout2), lambda n: (0, 0)),
            pl.BlockSpec(memory_space=pltpu.MemorySpace.SMEM),
            pl.BlockSpec(memory_space=pltpu.MemorySpace.SMEM),
        ],
        out_specs=pl.BlockSpec((B, NB, T2 * Cout2), lambda n: (0, n, 0)),
        out_shape=jax.ShapeDtypeStruct((B, N, T2 * Cout2), jnp.float32),
        compiler_params=pltpu.CompilerParams(dimension_semantics=("parallel",)),
    )(t2, w2b, b2, gamma, beta)
    return out.reshape(B, N, T2, Cout2)


def kernel(x, a_hat, w11, b11, w12, b12, w13, b13,
           w21, b21, w22, b22, w23, b23, theta, gamma, beta):
    K1, Cin, Cout = w11.shape
    K2, Csp, Cout2 = w21.shape
    w1m = jnp.concatenate([w11 + w13, w12], axis=-1).reshape(K1 * Cin, 2 * Cout)
    b1 = jnp.concatenate([b11 + b13, b12], axis=-1)
    w2m = jnp.concatenate([w21 + w23, w22], axis=-1).reshape(K2 * Csp, 2 * Cout2)
    b2 = jnp.concatenate([b21 + b23, b22], axis=-1)
    return _stgcn_forward(x, a_hat, w1m, b1, w2m, b2, theta, gamma, beta)


# speedup vs baseline: 2.0517x; 1.0301x over previous
import jax
import jax.numpy as jnp
from jax.experimental import pallas as pl
from jax.experimental.pallas import tpu as pltpu


def _stgcn_forward(x, a_hat, w1m, b1, w2m, b2, theta, gamma, beta):
    B, N, T, Cin = x.shape
    K1 = 3
    T1 = T - K1 + 1
    Cout = w1m.shape[-1] // 2
    Csp = theta.shape[1]
    K2 = 3
    T2 = T1 - K2 + 1
    Cout2 = w2m.shape[1] // 2

    w1b = w1m.astype(jnp.bfloat16)
    w2b = w2m.astype(jnp.bfloat16)
    ab = a_hat.astype(jnp.bfloat16)
    th2 = jnp.zeros((2 * Cout, 2 * Csp), theta.dtype)
    th2 = th2.at[:Cout, :Csp].set(theta).at[Cout:, Csp:].set(theta)
    th2b = th2.astype(jnp.bfloat16)

    BB = 4
    while B % BB:
        BB //= 2

    def k1(x_ref, w_ref, b_ref, th_ref, a_ref, o_ref):
        xk = x_ref[...].reshape(BB * N, T * Cin).astype(jnp.bfloat16)
        t_parts = []
        for t in range(T1):
            xs = xk[:, t * Cin:(t + K1) * Cin]
            y = jnp.dot(xs, w_ref[...],
                        preferred_element_type=jnp.float32) + b_ref[...]
            tt = jnp.maximum(y[:, :Cout] + jax.nn.sigmoid(y[:, Cout:]), 0.0)
            t_parts.append(tt.astype(jnp.bfloat16))
        u_parts = []
        for p in range(0, T1, 2):
            pair = jnp.concatenate(t_parts[p:p + 2], axis=-1)
            u_parts.append(jnp.dot(pair, th_ref[...],
                                   preferred_element_type=jnp.float32))
        u_all = jnp.concatenate(u_parts, axis=-1)
        u2 = u_all.astype(jnp.bfloat16).reshape(BB, N, T1 * Csp)
        for i in range(BB):
            lfs = jnp.dot(a_ref[...], u2[i],
                          preferred_element_type=jnp.float32)
            o_ref[i] = jnp.maximum(lfs, 0.0).astype(jnp.bfloat16)

    t2 = pl.pallas_call(
        k1,
        grid=(B // BB,),
        in_specs=[
            pl.BlockSpec((BB, N, T * Cin), lambda b: (b, 0, 0)),
            pl.BlockSpec((K1 * Cin, 2 * Cout), lambda b: (0, 0)),
            pl.BlockSpec((1, 2 * Cout), lambda b: (0, 0)),
            pl.BlockSpec((2 * Cout, 2 * Csp), lambda b: (0, 0)),
            pl.BlockSpec((N, N), lambda b: (0, 0)),
        ],
        out_specs=pl.BlockSpec((BB, N, T1 * Csp), lambda b: (b, 0, 0)),
        out_shape=jax.ShapeDtypeStruct((B, N, T1 * Csp), jnp.bfloat16),
        compiler_params=pltpu.CompilerParams(dimension_semantics=("parallel",)),
    )(x.reshape(B, N, T * Cin), w1b, b1, th2b, ab)

    NB = 16
    while N % NB:
        NB //= 2

    def k2(t2_ref, w_ref, b_ref, g_ref, bt_ref, o_ref):
        n0 = pl.program_id(0)
        z = t2_ref[...].reshape(B * NB, T1 * Csp)
        parts = []
        for t in range(T2):
            zs = z[:, t * Csp:(t + K2) * Csp]
            y = jnp.dot(zs, w_ref[...],
                        preferred_element_type=jnp.float32) + b_ref[...]
            parts.append(
                jnp.maximum(y[:, :Cout2] + jax.nn.sigmoid(y[:, Cout2:]), 0.0))
        t3 = jnp.concatenate(parts, axis=-1)
        t4 = t3.reshape(B, NB, T2 * Cout2)
        m = float(B * T2 * Cout2)
        mean = jnp.sum(t4, axis=(0, 2), keepdims=True) / m
        msq = jnp.sum(t4 * t4, axis=(0, 2), keepdims=True) / m
        inv = jax.lax.rsqrt(msq - mean * mean + 1e-5)
        gvec = jnp.stack([g_ref[n0 * NB + i] for i in range(NB)])
        bvec = jnp.stack([bt_ref[n0 * NB + i] for i in range(NB)])
        scale = inv * gvec.reshape(1, NB, 1)
        shift = bvec.reshape(1, NB, 1) - mean * scale
        o_ref[...] = t4 * scale + shift

    out = pl.pallas_call(
        k2,
        grid=(N // NB,),
        in_specs=[
            pl.BlockSpec((B, NB, T1 * Csp), lambda n: (0, n, 0)),
            pl.BlockSpec((K2 * Csp, 2 * Cout2), lambda n: (0, 0)),
            pl.BlockSpec((1, 2 * Cout2), lambda n: (0, 0)),
            pl.BlockSpec(memory_space=pltpu.MemorySpace.SMEM),
            pl.BlockSpec(memory_space=pltpu.MemorySpace.SMEM),
        ],
        out_specs=pl.BlockSpec((B, NB, T2 * Cout2), lambda n: (0, n, 0)),
        out_shape=jax.ShapeDtypeStruct((B, N, T2 * Cout2), jnp.float32),
        compiler_params=pltpu.CompilerParams(dimension_semantics=("parallel",)),
    )(t2, w2b, b2, gamma, beta)
    return out.reshape(B, N, T2, Cout2)


def kernel(x, a_hat, w11, b11, w12, b12, w13, b13,
           w21, b21, w22, b22, w23, b23, theta, gamma, beta):
    K1, Cin, Cout = w11.shape
    K2, Csp, Cout2 = w21.shape
    w1m = jnp.concatenate([w11 + w13, w12], axis=-1).reshape(K1 * Cin, 2 * Cout)
    b1 = jnp.concatenate([b11 + b13, b12], axis=-1)
    w2m = jnp.concatenate([w21 + w23, w22], axis=-1).reshape(K2 * Csp, 2 * Cout2)
    b2 = jnp.concatenate([b21 + b23, b22], axis=-1)
    return _stgcn_forward(x, a_hat, w1m, b1, w2m, b2, theta, gamma, beta)


# speedup vs baseline: 2.0876x; 1.0175x over previous
import jax
import jax.numpy as jnp
from jax.experimental import pallas as pl
from jax.experimental.pallas import tpu as pltpu


def _stgcn_forward(x, a_hat, w1m, b1, w2m, b2, theta, gamma, beta):
    B, N, T, Cin = x.shape
    K1 = 3
    T1 = T - K1 + 1
    Cout = w1m.shape[-1] // 2
    Csp = theta.shape[1]
    K2 = 3
    T2 = T1 - K2 + 1
    Cout2 = w2m.shape[1] // 2

    w1b = w1m.astype(jnp.bfloat16)
    w2b = w2m.astype(jnp.bfloat16)
    ab = a_hat.astype(jnp.bfloat16)
    th2 = jnp.zeros((2 * Cout, 2 * Csp), theta.dtype)
    th2 = th2.at[:Cout, :Csp].set(theta).at[Cout:, Csp:].set(theta)
    th2b = th2.astype(jnp.bfloat16)

    BB = 4
    while B % BB:
        BB //= 2

    def k1(x_ref, w_ref, b_ref, th_ref, a_ref, o_ref):
        xk = x_ref[...].reshape(BB * N, T * Cin).astype(jnp.bfloat16)
        t_parts = []
        for t in range(T1):
            xs = xk[:, t * Cin:(t + K1) * Cin]
            y = jnp.dot(xs, w_ref[...],
                        preferred_element_type=jnp.float32) + b_ref[...]
            tt = jnp.maximum(y[:, :Cout] + jax.nn.sigmoid(y[:, Cout:]), 0.0)
            t_parts.append(tt.astype(jnp.bfloat16))
        u_parts = []
        for p in range(0, T1, 2):
            pair = jnp.concatenate(t_parts[p:p + 2], axis=-1)
            u_parts.append(jnp.dot(pair, th_ref[...],
                                   preferred_element_type=jnp.float32))
        u_all = jnp.concatenate(u_parts, axis=-1)
        u2 = u_all.astype(jnp.bfloat16).reshape(BB, N, T1 * Csp)
        for i in range(BB):
            lfs = jnp.dot(a_ref[...], u2[i],
                          preferred_element_type=jnp.float32)
            o_ref[i] = jnp.maximum(lfs, 0.0).astype(jnp.bfloat16)

    t2 = pl.pallas_call(
        k1,
        grid=(B // BB,),
        in_specs=[
            pl.BlockSpec((BB, N, T * Cin), lambda b: (b, 0, 0)),
            pl.BlockSpec((K1 * Cin, 2 * Cout), lambda b: (0, 0)),
            pl.BlockSpec((1, 2 * Cout), lambda b: (0, 0)),
            pl.BlockSpec((2 * Cout, 2 * Csp), lambda b: (0, 0)),
            pl.BlockSpec((N, N), lambda b: (0, 0)),
        ],
        out_specs=pl.BlockSpec((BB, N, T1 * Csp), lambda b: (b, 0, 0)),
        out_shape=jax.ShapeDtypeStruct((B, N, T1 * Csp), jnp.bfloat16),
        compiler_params=pltpu.CompilerParams(dimension_semantics=("parallel",)),
    )(x.reshape(B, N, T * Cin), w1b, b1, th2b, ab)

    NB = 16
    while N % NB:
        NB //= 2

    def k2(t2_ref, w_ref, b_ref, g_ref, bt_ref, o_ref):
        n0 = pl.program_id(0)
        z = t2_ref[...].reshape(B * NB, T1 * Csp)
        parts = []
        s1 = jnp.zeros((1, NB, Cout2), jnp.float32)
        s2 = jnp.zeros((1, NB, Cout2), jnp.float32)
        for t in range(T2):
            zs = z[:, t * Csp:(t + K2) * Csp]
            y = jnp.dot(zs, w_ref[...],
                        preferred_element_type=jnp.float32) + b_ref[...]
            p = jnp.maximum(y[:, :Cout2] + jax.nn.sigmoid(y[:, Cout2:]), 0.0)
            parts.append(p)
            p3 = p.reshape(B, NB, Cout2)
            s1 = s1 + jnp.sum(p3, axis=0, keepdims=True)
            s2 = s2 + jnp.sum(p3 * p3, axis=0, keepdims=True)
        t3 = jnp.concatenate(parts, axis=-1)
        t4 = t3.reshape(B, NB, T2 * Cout2)
        m = float(B * T2 * Cout2)
        mean = jnp.sum(s1, axis=2, keepdims=True) / m
        msq = jnp.sum(s2, axis=2, keepdims=True) / m
        inv = jax.lax.rsqrt(msq - mean * mean + 1e-5)
        gvec = jnp.stack([g_ref[n0 * NB + i] for i in range(NB)])
        bvec = jnp.stack([bt_ref[n0 * NB + i] for i in range(NB)])
        scale = inv * gvec.reshape(1, NB, 1)
        shift = bvec.reshape(1, NB, 1) - mean * scale
        o_ref[...] = t4 * scale + shift

    out = pl.pallas_call(
        k2,
        grid=(N // NB,),
        in_specs=[
            pl.BlockSpec((B, NB, T1 * Csp), lambda n: (0, n, 0)),
            pl.BlockSpec((K2 * Csp, 2 * Cout2), lambda n: (0, 0)),
            pl.BlockSpec((1, 2 * Cout2), lambda n: (0, 0)),
            pl.BlockSpec(memory_space=pltpu.MemorySpace.SMEM),
            pl.BlockSpec(memory_space=pltpu.MemorySpace.SMEM),
        ],
        out_specs=pl.BlockSpec((B, NB, T2 * Cout2), lambda n: (0, n, 0)),
        out_shape=jax.ShapeDtypeStruct((B, N, T2 * Cout2), jnp.float32),
        compiler_params=pltpu.CompilerParams(dimension_semantics=("parallel",)),
    )(t2, w2b, b2, gamma, beta)
    return out.reshape(B, N, T2, Cout2)


def kernel(x, a_hat, w11, b11, w12, b12, w13, b13,
           w21, b21, w22, b22, w23, b23, theta, gamma, beta):
    K1, Cin, Cout = w11.shape
    K2, Csp, Cout2 = w21.shape
    w1m = jnp.concatenate([w11 + w13, w12], axis=-1).reshape(K1 * Cin, 2 * Cout)
    b1 = jnp.concatenate([b11 + b13, b12], axis=-1)
    w2m = jnp.concatenate([w21 + w23, w22], axis=-1).reshape(K2 * Csp, 2 * Cout2)
    b2 = jnp.concatenate([b21 + b23, b22], axis=-1)
    return _stgcn_forward(x, a_hat, w1m, b1, w2m, b2, theta, gamma, beta)


# speedup vs baseline: 2.1509x; 1.0303x over previous
import jax
import jax.numpy as jnp
from jax.experimental import pallas as pl
from jax.experimental.pallas import tpu as pltpu


def _stgcn_fused(x, a_hat, w1m, b1, w2m, b2, theta, gamma, beta):
    B, N, T, Cin = x.shape
    K1 = 3
    T1 = T - K1 + 1
    Cout = w1m.shape[-1] // 2
    Csp = theta.shape[1]
    K2 = 3
    T2 = T1 - K2 + 1
    Cout2 = w2m.shape[1] // 2

    w1b = w1m.astype(jnp.bfloat16)
    w2b = w2m.astype(jnp.bfloat16)
    ab = a_hat.astype(jnp.bfloat16)
    th2 = jnp.zeros((2 * Cout, 2 * Csp), theta.dtype)
    th2 = th2.at[:Cout, :Csp].set(theta).at[Cout:, Csp:].set(theta)
    th2b = th2.astype(jnp.bfloat16)

    BB = 4
    while B % BB:
        BB //= 2
    NB = 16
    while N % NB:
        NB //= 2
    G1 = B // BB
    G2 = N // NB

    def body(x_ref, w1_ref, b1_ref, th_ref, a_ref, w2_ref, b2_ref,
             g_ref, bt_ref, o_ref, t2s):
        i = pl.program_id(0)

        @pl.when(i < G1)
        def _phase1():
            xk = x_ref[...].reshape(BB * N, T * Cin).astype(jnp.bfloat16)
            t_parts = []
            for t in range(T1):
                xs = xk[:, t * Cin:(t + K1) * Cin]
                y = jnp.dot(xs, w1_ref[...],
                            preferred_element_type=jnp.float32) + b1_ref[...]
                tt = jnp.maximum(y[:, :Cout] + jax.nn.sigmoid(y[:, Cout:]),
                                 0.0)
                t_parts.append(tt.astype(jnp.bfloat16))
            u_parts = []
            for p in range(0, T1, 2):
                pair = jnp.concatenate(t_parts[p:p + 2], axis=-1)
                u_parts.append(jnp.dot(pair, th_ref[...],
                                       preferred_element_type=jnp.float32))
            u_all = jnp.concatenate(u_parts, axis=-1)
            u2 = u_all.astype(jnp.bfloat16).reshape(BB, N, T1 * Csp)
            for j in range(BB):
                lfs = jnp.dot(a_ref[...], u2[j],
                              preferred_element_type=jnp.float32)
                t2s[i * BB + j] = jnp.maximum(lfs, 0.0).astype(jnp.bfloat16)

        @pl.when(i >= G1)
        def _phase2():
            n0 = (i - G1) * NB
            z = t2s[:, pl.ds(n0, NB), :].reshape(B * NB, T1 * Csp)
            parts = []
            s1 = jnp.zeros((1, NB, Cout2), jnp.float32)
            s2 = jnp.zeros((1, NB, Cout2), jnp.float32)
            for t in range(T2):
                zs = z[:, t * Csp:(t + K2) * Csp]
                y = jnp.dot(zs, w2_ref[...],
                            preferred_element_type=jnp.float32) + b2_ref[...]
                p = jnp.maximum(y[:, :Cout2] + jax.nn.sigmoid(y[:, Cout2:]),
                                0.0)
                parts.append(p)
                p3 = p.reshape(B, NB, Cout2)
                s1 = s1 + jnp.sum(p3, axis=0, keepdims=True)
                s2 = s2 + jnp.sum(p3 * p3, axis=0, keepdims=True)
            t3 = jnp.concatenate(parts, axis=-1)
            t4 = t3.reshape(B, NB, T2 * Cout2)
            m = float(B * T2 * Cout2)
            mean = jnp.sum(s1, axis=2, keepdims=True) / m
            msq = jnp.sum(s2, axis=2, keepdims=True) / m
            inv = jax.lax.rsqrt(msq - mean * mean + 1e-5)
            gvec = jnp.stack([g_ref[n0 + k] for k in range(NB)])
            bvec = jnp.stack([bt_ref[n0 + k] for k in range(NB)])
            scale = inv * gvec.reshape(1, NB, 1)
            shift = bvec.reshape(1, NB, 1) - mean * scale
            o_ref[...] = t4 * scale + shift

    out = pl.pallas_call(
        body,
        grid=(G1 + G2,),
        in_specs=[
            pl.BlockSpec((BB, N, T * Cin),
                         lambda i: (jnp.minimum(i, G1 - 1), 0, 0)),
            pl.BlockSpec((K1 * Cin, 2 * Cout), lambda i: (0, 0)),
            pl.BlockSpec((1, 2 * Cout), lambda i: (0, 0)),
            pl.BlockSpec((2 * Cout, 2 * Csp), lambda i: (0, 0)),
            pl.BlockSpec((N, N), lambda i: (0, 0)),
            pl.BlockSpec((K2 * Csp, 2 * Cout2), lambda i: (0, 0)),
            pl.BlockSpec((1, 2 * Cout2), lambda i: (0, 0)),
            pl.BlockSpec(memory_space=pltpu.MemorySpace.SMEM),
            pl.BlockSpec(memory_space=pltpu.MemorySpace.SMEM),
        ],
        out_specs=pl.BlockSpec((B, NB, T2 * Cout2),
                               lambda i: (0, jnp.maximum(i - G1, 0), 0)),
        out_shape=jax.ShapeDtypeStruct((B, N, T2 * Cout2), jnp.float32),
        scratch_shapes=[pltpu.VMEM((B, N, T1 * Csp), jnp.bfloat16)],
        compiler_params=pltpu.CompilerParams(
            dimension_semantics=("arbitrary",)),
    )(x.reshape(B, N, T * Cin), w1b, b1, th2b, ab, w2b, b2, gamma, beta)
    return out.reshape(B, N, T2, Cout2)


def kernel(x, a_hat, w11, b11, w12, b12, w13, b13,
           w21, b21, w22, b22, w23, b23, theta, gamma, beta):
    K1, Cin, Cout = w11.shape
    K2, Csp, Cout2 = w21.shape
    w1m = jnp.concatenate([w11 + w13, w12], axis=-1).reshape(K1 * Cin, 2 * Cout)
    b1 = jnp.concatenate([b11 + b13, b12], axis=-1)
    w2m = jnp.concatenate([w21 + w23, w22], axis=-1).reshape(K2 * Csp, 2 * Cout2)
    b2 = jnp.concatenate([b21 + b23, b22], axis=-1)
    return _stgcn_fused(x, a_hat, w1m, b1, w2m, b2, theta, gamma, beta)


# speedup vs baseline: 2.2064x; 1.0258x over previous
import jax
import jax.numpy as jnp
from jax.experimental import pallas as pl
from jax.experimental.pallas import tpu as pltpu


def _stgcn_fused(x, a_hat, w1m, b1, w2m, b2, theta, gamma, beta):
    B, N, T, Cin = x.shape
    K1 = 3
    T1 = T - K1 + 1
    Cout = w1m.shape[-1] // 2
    Csp = theta.shape[1]
    K2 = 3
    T2 = T1 - K2 + 1
    Cout2 = w2m.shape[1] // 2

    w1b = w1m.astype(jnp.bfloat16)
    w2b = w2m.astype(jnp.bfloat16)
    ab = a_hat.astype(jnp.bfloat16)
    th2 = jnp.zeros((2 * Cout, 2 * Csp), theta.dtype)
    th2 = th2.at[:Cout, :Csp].set(theta).at[Cout:, Csp:].set(theta)
    th2b = th2.astype(jnp.bfloat16)

    BB = 8
    while B % BB:
        BB //= 2
    NB = 16
    while N % NB:
        NB //= 2
    G1 = B // BB
    G2 = N // NB

    def body(x_ref, w1_ref, b1_ref, th_ref, a_ref, w2_ref, b2_ref,
             g_ref, bt_ref, o_ref, t2s):
        i = pl.program_id(0)

        @pl.when(i < G1)
        def _phase1():
            xk = x_ref[...].reshape(BB * N, T * Cin).astype(jnp.bfloat16)
            t_parts = []
            for t in range(T1):
                xs = xk[:, t * Cin:(t + K1) * Cin]
                y = jnp.dot(xs, w1_ref[...],
                            preferred_element_type=jnp.float32) + b1_ref[...]
                tt = jnp.maximum(y[:, :Cout] + jax.nn.sigmoid(y[:, Cout:]),
                                 0.0)
                t_parts.append(tt.astype(jnp.bfloat16))
            u_parts = []
            for p in range(0, T1, 2):
                pair = jnp.concatenate(t_parts[p:p + 2], axis=-1)
                u_parts.append(jnp.dot(pair, th_ref[...],
                                       preferred_element_type=jnp.float32))
            u_all = jnp.concatenate(u_parts, axis=-1)
            u2 = u_all.astype(jnp.bfloat16).reshape(BB, N, T1 * Csp)
            for j in range(BB):
                lfs = jnp.dot(a_ref[...], u2[j],
                              preferred_element_type=jnp.float32)
                t2s[i * BB + j] = jnp.maximum(lfs, 0.0).astype(jnp.bfloat16)

        @pl.when(i >= G1)
        def _phase2():
            n0 = (i - G1) * NB
            z = t2s[:, pl.ds(n0, NB), :].reshape(B * NB, T1 * Csp)
            parts = []
            s1 = jnp.zeros((1, NB, Cout2), jnp.float32)
            s2 = jnp.zeros((1, NB, Cout2), jnp.float32)
            for t in range(T2):
                zs = z[:, t * Csp:(t + K2) * Csp]
                y = jnp.dot(zs, w2_ref[...],
                            preferred_element_type=jnp.float32) + b2_ref[...]
                p = jnp.maximum(y[:, :Cout2] + jax.nn.sigmoid(y[:, Cout2:]),
                                0.0)
                parts.append(p)
                p3 = p.reshape(B, NB, Cout2)
                s1 = s1 + jnp.sum(p3, axis=0, keepdims=True)
                s2 = s2 + jnp.sum(p3 * p3, axis=0, keepdims=True)
            t3 = jnp.concatenate(parts, axis=-1)
            t4 = t3.reshape(B, NB, T2 * Cout2)
            m = float(B * T2 * Cout2)
            mean = jnp.sum(s1, axis=2, keepdims=True) / m
            msq = jnp.sum(s2, axis=2, keepdims=True) / m
            inv = jax.lax.rsqrt(msq - mean * mean + 1e-5)
            gvec = jnp.stack([g_ref[n0 + k] for k in range(NB)])
            bvec = jnp.stack([bt_ref[n0 + k] for k in range(NB)])
            scale = inv * gvec.reshape(1, NB, 1)
            shift = bvec.reshape(1, NB, 1) - mean * scale
            o_ref[...] = t4 * scale + shift

    out = pl.pallas_call(
        body,
        grid=(G1 + G2,),
        in_specs=[
            pl.BlockSpec((BB, N, T * Cin),
                         lambda i: (jnp.minimum(i, G1 - 1), 0, 0)),
            pl.BlockSpec((K1 * Cin, 2 * Cout), lambda i: (0, 0)),
            pl.BlockSpec((1, 2 * Cout), lambda i: (0, 0)),
            pl.BlockSpec((2 * Cout, 2 * Csp), lambda i: (0, 0)),
            pl.BlockSpec((N, N), lambda i: (0, 0)),
            pl.BlockSpec((K2 * Csp, 2 * Cout2), lambda i: (0, 0)),
            pl.BlockSpec((1, 2 * Cout2), lambda i: (0, 0)),
            pl.BlockSpec(memory_space=pltpu.MemorySpace.SMEM),
            pl.BlockSpec(memory_space=pltpu.MemorySpace.SMEM),
        ],
        out_specs=pl.BlockSpec((B, NB, T2 * Cout2),
                               lambda i: (0, jnp.maximum(i - G1, 0), 0)),
        out_shape=jax.ShapeDtypeStruct((B, N, T2 * Cout2), jnp.float32),
        scratch_shapes=[pltpu.VMEM((B, N, T1 * Csp), jnp.bfloat16)],
        compiler_params=pltpu.CompilerParams(
            dimension_semantics=("arbitrary",)),
    )(x.reshape(B, N, T * Cin), w1b, b1, th2b, ab, w2b, b2, gamma, beta)
    return out.reshape(B, N, T2, Cout2)


def kernel(x, a_hat, w11, b11, w12, b12, w13, b13,
           w21, b21, w22, b22, w23, b23, theta, gamma, beta):
    K1, Cin, Cout = w11.shape
    K2, Csp, Cout2 = w21.shape
    w1m = jnp.concatenate([w11 + w13, w12], axis=-1).reshape(K1 * Cin, 2 * Cout)
    b1 = jnp.concatenate([b11 + b13, b12], axis=-1)
    w2m = jnp.concatenate([w21 + w23, w22], axis=-1).reshape(K2 * Csp, 2 * Cout2)
    b2 = jnp.concatenate([b21 + b23, b22], axis=-1)
    return _stgcn_fused(x, a_hat, w1m, b1, w2m, b2, theta, gamma, beta)


# speedup vs baseline: 2.2139x; 1.0034x over previous
import jax
import jax.numpy as jnp
from jax.experimental import pallas as pl
from jax.experimental.pallas import tpu as pltpu


def _stgcn_fused(x, a_hat, w1m, b1, w2m, b2, theta, gamma, beta):
    B, N, T, Cin = x.shape
    K1 = 3
    T1 = T - K1 + 1
    Cout = w1m.shape[-1] // 2
    Csp = theta.shape[1]
    K2 = 3
    T2 = T1 - K2 + 1
    Cout2 = w2m.shape[1] // 2

    w1b = w1m.astype(jnp.bfloat16)
    w2b = w2m.astype(jnp.bfloat16)
    ab = a_hat.astype(jnp.bfloat16)
    th2 = jnp.zeros((2 * Cout, 2 * Csp), theta.dtype)
    th2 = th2.at[:Cout, :Csp].set(theta).at[Cout:, Csp:].set(theta)
    th2b = th2.astype(jnp.bfloat16)

    BB = 16
    while B % BB:
        BB //= 2
    NB = 16
    while N % NB:
        NB //= 2
    G1 = B // BB
    G2 = N // NB

    def body(x_ref, w1_ref, b1_ref, th_ref, a_ref, w2_ref, b2_ref,
             g_ref, bt_ref, o_ref, t2s):
        i = pl.program_id(0)

        @pl.when(i < G1)
        def _phase1():
            xk = x_ref[...].reshape(BB * N, T * Cin).astype(jnp.bfloat16)
            t_parts = []
            for t in range(T1):
                xs = xk[:, t * Cin:(t + K1) * Cin]
                y = jnp.dot(xs, w1_ref[...],
                            preferred_element_type=jnp.float32) + b1_ref[...]
                tt = jnp.maximum(y[:, :Cout] + jax.nn.sigmoid(y[:, Cout:]),
                                 0.0)
                t_parts.append(tt.astype(jnp.bfloat16))
            u_parts = []
            for p in range(0, T1, 2):
                pair = jnp.concatenate(t_parts[p:p + 2], axis=-1)
                u_parts.append(jnp.dot(pair, th_ref[...],
                                       preferred_element_type=jnp.float32))
            u_all = jnp.concatenate(u_parts, axis=-1)
            u2 = u_all.astype(jnp.bfloat16).reshape(BB, N, T1 * Csp)
            for j in range(BB):
                lfs = jnp.dot(a_ref[...], u2[j],
                              preferred_element_type=jnp.float32)
                t2s[i * BB + j] = jnp.maximum(lfs, 0.0).astype(jnp.bfloat16)

        @pl.when(i >= G1)
        def _phase2():
            n0 = (i - G1) * NB
            z = t2s[:, pl.ds(n0, NB), :].reshape(B * NB, T1 * Csp)
            parts = []
            s1 = jnp.zeros((1, NB, Cout2), jnp.float32)
            s2 = jnp.zeros((1, NB, Cout2), jnp.float32)
            for t in range(T2):
                zs = z[:, t * Csp:(t + K2) * Csp]
                y = jnp.dot(zs, w2_ref[...],
                            preferred_element_type=jnp.float32) + b2_ref[...]
                p = jnp.maximum(y[:, :Cout2] + jax.nn.sigmoid(y[:, Cout2:]),
                                0.0)
                parts.append(p)
                p3 = p.reshape(B, NB, Cout2)
                s1 = s1 + jnp.sum(p3, axis=0, keepdims=True)
                s2 = s2 + jnp.sum(p3 * p3, axis=0, keepdims=True)
            t3 = jnp.concatenate(parts, axis=-1)
            t4 = t3.reshape(B, NB, T2 * Cout2)
            m = float(B * T2 * Cout2)
            mean = jnp.sum(s1, axis=2, keepdims=True) / m
            msq = jnp.sum(s2, axis=2, keepdims=True) / m
            inv = jax.lax.rsqrt(msq - mean * mean + 1e-5)
            gvec = jnp.stack([g_ref[n0 + k] for k in range(NB)])
            bvec = jnp.stack([bt_ref[n0 + k] for k in range(NB)])
            scale = inv * gvec.reshape(1, NB, 1)
            shift = bvec.reshape(1, NB, 1) - mean * scale
            o_ref[...] = t4 * scale + shift

    out = pl.pallas_call(
        body,
        grid=(G1 + G2,),
        in_specs=[
            pl.BlockSpec((BB, N, T * Cin),
                         lambda i: (jnp.minimum(i, G1 - 1), 0, 0)),
            pl.BlockSpec((K1 * Cin, 2 * Cout), lambda i: (0, 0)),
            pl.BlockSpec((1, 2 * Cout), lambda i: (0, 0)),
            pl.BlockSpec((2 * Cout, 2 * Csp), lambda i: (0, 0)),
            pl.BlockSpec((N, N), lambda i: (0, 0)),
            pl.BlockSpec((K2 * Csp, 2 * Cout2), lambda i: (0, 0)),
            pl.BlockSpec((1, 2 * Cout2), lambda i: (0, 0)),
            pl.BlockSpec(memory_space=pltpu.MemorySpace.SMEM),
            pl.BlockSpec(memory_space=pltpu.MemorySpace.SMEM),
        ],
        out_specs=pl.BlockSpec((B, NB, T2 * Cout2),
                               lambda i: (0, jnp.maximum(i - G1, 0), 0)),
        out_shape=jax.ShapeDtypeStruct((B, N, T2 * Cout2), jnp.float32),
        scratch_shapes=[pltpu.VMEM((B, N, T1 * Csp), jnp.bfloat16)],
        compiler_params=pltpu.CompilerParams(
            dimension_semantics=("arbitrary",)),
    )(x.reshape(B, N, T * Cin), w1b, b1, th2b, ab, w2b, b2, gamma, beta)
    return out.reshape(B, N, T2, Cout2)


def kernel(x, a_hat, w11, b11, w12, b12, w13, b13,
           w21, b21, w22, b22, w23, b23, theta, gamma, beta):
    K1, Cin, Cout = w11.shape
    K2, Csp, Cout2 = w21.shape
    w1m = jnp.concatenate([w11 + w13, w12], axis=-1).reshape(K1 * Cin, 2 * Cout)
    b1 = jnp.concatenate([b11 + b13, b12], axis=-1)
    w2m = jnp.concatenate([w21 + w23, w22], axis=-1).reshape(K2 * Csp, 2 * Cout2)
    b2 = jnp.concatenate([b21 + b23, b22], axis=-1)
    return _stgcn_fused(x, a_hat, w1m, b1, w2m, b2, theta, gamma, beta)
